# baseline (speedup 1.0000x reference)
import jax
import jax.numpy as jnp
from jax import lax
from jax.experimental import pallas as pl
from jax.experimental.pallas import tpu as pltpu
from jax.experimental.pallas import tpu_sc as plsc

_N = 100000
_NPAD = 100096
_MAX_DEG = 16
_WALK_LEN = 16
_NUM_CORES = 2
_NUM_SUBCORES = 16
_NW = _NUM_CORES * _NUM_SUBCORES
_LANES = 16
_CHUNK = 3200
_TAIL = _NPAD - (_NW - 1) * _CHUNK
_NVEC = _CHUNK // _LANES
_NSPLIT = 2
_Q = _CHUNK // _NSPLIT
_NQ = _NVEC // _NSPLIT


def _walk_body(neigh_hbm, degp_hbm, unif_hbm, out_hbm,
               degp_v, cur_v, flat_v, d0_v, gath_v, u_a, u_b, o_st0, o_st1,
               sem_deg, sem_ua, sem_ub, sem_g0, sem_g1, sem_o0, sem_o1):
    wid = lax.axis_index("s") * _NUM_CORES + lax.axis_index("c")
    base = pl.multiple_of(wid * _CHUNK, 128)
    last = wid == _NW - 1
    sem_g = (sem_g0, sem_g1)
    o_sts = (o_st0, o_st1)
    sem_os = (sem_o0, sem_o1)

    cp_deg = pltpu.async_copy(degp_hbm, degp_v, sem_deg)

    _UTAIL = _N - (_NW - 1) * _CHUNK

    def fire_u(t):
        u_ref, u_sem = (u_a, sem_ua) if t % 2 == 0 else (u_b, sem_ub)
        off = pl.multiple_of(t * _N + base, _LANES)

        @pl.when(jnp.logical_not(last))
        def _():
            pltpu.async_copy(unif_hbm.at[pl.ds(off, _CHUNK)], u_ref, u_sem)

        @pl.when(last)
        def _():
            pltpu.async_copy(unif_hbm.at[pl.ds(off, _UTAIL)],
                             u_ref.at[pl.ds(0, _UTAIL)], u_sem)

    def wait_u(t):
        u_ref, u_sem = (u_a, sem_ua) if t % 2 == 0 else (u_b, sem_ub)
        off = pl.multiple_of(t * _N + base, _LANES)

        @pl.when(jnp.logical_not(last))
        def _():
            pltpu.make_async_copy(unif_hbm.at[pl.ds(off, _CHUNK)], u_ref,
                                  u_sem).wait()

        @pl.when(last)
        def _():
            pltpu.make_async_copy(unif_hbm.at[pl.ds(off, _UTAIL)],
                                  u_ref.at[pl.ds(0, _UTAIL)], u_sem).wait()

    fire_u(0)
    fire_u(1)
    cp_deg.wait()

    def fire_out(g):
        @pl.when(jnp.logical_not(last))
        def _():
            pltpu.async_copy(o_sts[g], out_hbm.at[g, :, pl.ds(base, _CHUNK)],
                             sem_os[g])

        @pl.when(last)
        def _():
            pltpu.async_copy(o_sts[g].at[:, pl.ds(0, _TAIL)],
                             out_hbm.at[g, :, pl.ds(base, _TAIL)], sem_os[g])

    def wait_out(g):
        @pl.when(jnp.logical_not(last))
        def _():
            pltpu.make_async_copy(o_sts[g],
                                  out_hbm.at[g, :, pl.ds(base, _CHUNK)],
                                  sem_os[g]).wait()

        @pl.when(last)
        def _():
            pltpu.make_async_copy(o_sts[g].at[:, pl.ds(0, _TAIL)],
                                  out_hbm.at[g, :, pl.ds(base, _TAIL)],
                                  sem_os[g]).wait()

    def fused_pass(t, q, u_ref):
        @plsc.parallel_loop(q * _NQ, (q + 1) * _NQ, unroll=2)
        def _f(j):
            sl = pl.ds(j * _LANES, _LANES)
            if t == 0:
                cur = jnp.minimum(
                    base + j * _LANES + lax.iota(jnp.int32, _LANES), _N - 1)
            else:
                cur = jnp.where(d0_v[sl] > 0, gath_v[sl], cur_v[sl])
                o_sts[(t - 1) // 8][(t - 1) % 8, sl] = cur
            cur_v[sl] = cur
            if t < _WALK_LEN:
                w = plsc.load_gather(degp_v,
                                     [lax.shift_right_logical(cur, 3)])
                sh = (cur & 7) * 4
                d0 = lax.shift_right_logical(w, sh) & 15
                d = jnp.maximum(d0, 1)
                y = d.astype(jnp.float32) * u_ref[sl]
                i = y.astype(jnp.int32)
                idx = jnp.where(i.astype(jnp.float32) < y, i, i - 1)
                idx = jnp.maximum(jnp.minimum(idx, d - 1), 0)
                flat_v[sl] = cur * _MAX_DEG + idx
                d0_v[sl] = d0

    g_descs = {}
    for t in range(_WALK_LEN + 1):
        u_ref = u_a if t % 2 == 0 else u_b
        if t < _WALK_LEN:
            wait_u(t)
        for q in range(_NSPLIT):
            qs = pl.ds(q * _Q, _Q)
            if t >= 1:
                g_descs[(t - 1, q)].wait()
            fused_pass(t, q, u_ref)
            if t < _WALK_LEN:
                g_descs[(t, q)] = pltpu.async_copy(
                    neigh_hbm.at[flat_v.at[qs]], gath_v.at[qs], sem_g[q])
        if t == 8:
            fire_out(0)
        if t + 2 <= _WALK_LEN - 1:
            fire_u(t + 2)
    fire_out(1)
    wait_out(0)
    wait_out(1)


@jax.jit
def kernel(neighbors, degrees, uniforms):
    mesh = plsc.VectorSubcoreMesh(core_axis_name="c", subcore_axis_name="s")
    walk = pl.kernel(
        _walk_body,
        out_type=jax.ShapeDtypeStruct((2, 8, _NPAD), jnp.int32),
        mesh=mesh,
        compiler_params=pltpu.CompilerParams(needs_layout_passes=False),
        scratch_types=[
            pltpu.VMEM((_N // 8,), jnp.int32),
            pltpu.VMEM((_CHUNK,), jnp.int32),
            pltpu.VMEM((_CHUNK,), jnp.int32),
            pltpu.VMEM((_CHUNK,), jnp.int32),
            pltpu.VMEM((_CHUNK,), jnp.int32),
            pltpu.VMEM((_CHUNK,), jnp.float32),
            pltpu.VMEM((_CHUNK,), jnp.float32),
            pltpu.VMEM((8, _CHUNK), jnp.int32),
            pltpu.VMEM((8, _CHUNK), jnp.int32),
            pltpu.SemaphoreType.DMA,
            pltpu.SemaphoreType.DMA,
            pltpu.SemaphoreType.DMA,
            pltpu.SemaphoreType.DMA,
            pltpu.SemaphoreType.DMA,
            pltpu.SemaphoreType.DMA,
            pltpu.SemaphoreType.DMA,
        ],
    )
    shifts = jnp.arange(8, dtype=jnp.uint32) * 4
    degp = (degrees.reshape(-1, 8).astype(jnp.uint32) << shifts).sum(
        axis=1, dtype=jnp.uint32).astype(jnp.int32)
    out3 = walk(neighbors.reshape(-1), degp, uniforms.reshape(-1))
    return out3[:, :, :_N].reshape(_WALK_LEN, _N)


# speedup vs baseline: 1.4648x; 1.4648x over previous
import jax
import jax.numpy as jnp
from jax import lax
from jax.experimental import pallas as pl
from jax.experimental.pallas import tpu as pltpu
from jax.experimental.pallas import tpu_sc as plsc

_N = 100000
_MAX_DEG = 16
_WALK_LEN = 16
_NUM_CORES = 2
_NUM_SUBCORES = 16
_LANES = 16
_CHUNK = 3136
_NVEC = _CHUNK // _LANES
_H = _CHUNK // 2
_NH = _NVEC // 2


def _walk_body(neigh_hbm, deg_hbm, unif_hbm, out_hbm,
               deg_v, cur_v, flat_v, d0_v, u_a, u_b, gath_v,
               sem_deg, sem_ua, sem_ub, sem_g0, sem_g1, sem_out):
    wid = lax.axis_index("s") * _NUM_CORES + lax.axis_index("c")
    base = jnp.minimum(wid * _CHUNK, _N - _CHUNK)

    cp_deg = pltpu.async_copy(deg_hbm, deg_v, sem_deg)

    @plsc.parallel_loop(0, _NVEC, unroll=4)
    def _init(j):
        cur_v[pl.ds(j * _LANES, _LANES)] = (
            base + j * _LANES + lax.iota(jnp.int32, _LANES))

    pltpu.async_copy(unif_hbm.at[pl.ds(pl.multiple_of(base, _LANES), _CHUNK)],
                     u_a, sem_ua)
    cp_deg.wait()

    def pick_half(u_ref, h):
        @plsc.parallel_loop(h * _NH, (h + 1) * _NH, unroll=2)
        def _pick(j):
            sl = pl.ds(j * _LANES, _LANES)
            cur = cur_v[sl]
            d0 = plsc.load_gather(deg_v, [cur])
            d = jnp.maximum(d0, 1)
            y = d.astype(jnp.float32) * u_ref[sl]
            i = y.astype(jnp.int32)
            idx = jnp.where(i.astype(jnp.float32) < y, i, i - 1)
            idx = jnp.maximum(jnp.minimum(idx, d - 1), 0)
            flat_v[sl] = cur * _MAX_DEG + idx
            d0_v[sl] = d0

    def sel_half(h):
        @plsc.parallel_loop(h * _NH, (h + 1) * _NH, unroll=2)
        def _sel(j):
            sl = pl.ds(j * _LANES, _LANES)
            g = gath_v[pl.ds(j * _LANES, _LANES)]
            cur_v[sl] = jnp.where(d0_v[sl] > 0, g, cur_v[sl])

    def one_step(t, u_ref, u_sem, first):
        pltpu.make_async_copy(
            unif_hbm.at[pl.ds(pl.multiple_of(base, _LANES), _CHUNK)],
            u_ref, u_sem).wait()
        pick_half(u_ref, 0)
        g0 = pltpu.async_copy(neigh_hbm.at[flat_v.at[pl.ds(0, _H)]],
                              gath_v.at[pl.ds(0, _H)], sem_g0)
        pick_half(u_ref, 1)
        g1 = pltpu.async_copy(neigh_hbm.at[flat_v.at[pl.ds(_H, _H)]],
                              gath_v.at[pl.ds(_H, _H)], sem_g1)
        g0.wait()
        if not first:
            pltpu.make_async_copy(
                cur_v,
                out_hbm.at[pl.ds(pl.multiple_of(base, _LANES), _CHUNK)],
                sem_out).wait()
        sel_half(0)
        g1.wait()
        sel_half(1)
        off = pl.multiple_of(t * _N + base, _LANES)
        pltpu.async_copy(cur_v, out_hbm.at[pl.ds(off, _CHUNK)], sem_out)

    def pair_body(k, carry):
        t0 = 2 * k
        t1 = 2 * k + 1
        off1 = pl.multiple_of(t1 * _N + base, _LANES)
        pltpu.async_copy(unif_hbm.at[pl.ds(off1, _CHUNK)], u_b, sem_ub)
        one_step(t0, u_a, sem_ua, False)

        @pl.when(k < _WALK_LEN // 2 - 1)
        def _():
            off2 = pl.multiple_of((t1 + 1) * _N + base, _LANES)
            pltpu.async_copy(unif_hbm.at[pl.ds(off2, _CHUNK)], u_a, sem_ua)

        one_step(t1, u_b, sem_ub, False)
        return carry

    off1 = pl.multiple_of(_N + base, _LANES)
    pltpu.async_copy(unif_hbm.at[pl.ds(off1, _CHUNK)], u_b, sem_ub)
    one_step(0, u_a, sem_ua, True)
    off2 = pl.multiple_of(2 * _N + base, _LANES)
    pltpu.async_copy(unif_hbm.at[pl.ds(off2, _CHUNK)], u_a, sem_ua)
    one_step(1, u_b, sem_ub, False)
    lax.fori_loop(1, _WALK_LEN // 2, pair_body, 0)

    pltpu.make_async_copy(
        cur_v, out_hbm.at[pl.ds(pl.multiple_of(base, _LANES), _CHUNK)],
        sem_out).wait()


@jax.jit
def kernel(neighbors, degrees, uniforms):
    mesh = plsc.VectorSubcoreMesh(core_axis_name="c", subcore_axis_name="s")
    walk = pl.kernel(
        _walk_body,
        out_type=jax.ShapeDtypeStruct((_WALK_LEN * _N,), jnp.int32),
        mesh=mesh,
        compiler_params=pltpu.CompilerParams(needs_layout_passes=False),
        scratch_types=[
            pltpu.VMEM((_N,), jnp.int32),
            pltpu.VMEM((_CHUNK,), jnp.int32),
            pltpu.VMEM((_CHUNK,), jnp.int32),
            pltpu.VMEM((_CHUNK,), jnp.int32),
            pltpu.VMEM((_CHUNK,), jnp.float32),
            pltpu.VMEM((_CHUNK,), jnp.float32),
            pltpu.VMEM((_CHUNK,), jnp.int32),
            pltpu.SemaphoreType.DMA,
            pltpu.SemaphoreType.DMA,
            pltpu.SemaphoreType.DMA,
            pltpu.SemaphoreType.DMA,
            pltpu.SemaphoreType.DMA,
            pltpu.SemaphoreType.DMA,
        ],
    )
    out = walk(neighbors.reshape(-1), degrees, uniforms.reshape(-1))
    return out.reshape(_WALK_LEN, _N)


# speedup vs baseline: 1.4672x; 1.0016x over previous
import jax
import jax.numpy as jnp
from jax import lax
from jax.experimental import pallas as pl
from jax.experimental.pallas import tpu as pltpu
from jax.experimental.pallas import tpu_sc as plsc

_N = 100000
_MAX_DEG = 16
_WALK_LEN = 16
_NUM_CORES = 2
_NUM_SUBCORES = 16
_LANES = 16
_CHUNK = 3136
_NVEC = _CHUNK // _LANES
_H = _CHUNK // 2
_NH = _NVEC // 2


def _walk_body(neigh_hbm, deg_hbm, unif_hbm, out_hbm,
               deg_v, cur_v, flat_v, d0_v, u_a, u_b, gath_v,
               sem_deg, sem_ua, sem_ub, sem_g0, sem_g1, sem_out):
    wid = lax.axis_index("s") * _NUM_CORES + lax.axis_index("c")
    base = jnp.minimum(wid * _CHUNK, _N - _CHUNK)

    cp_deg = pltpu.async_copy(deg_hbm, deg_v, sem_deg)

    @plsc.parallel_loop(0, _NVEC, unroll=4)
    def _init(j):
        cur_v[pl.ds(j * _LANES, _LANES)] = (
            base + j * _LANES + lax.iota(jnp.int32, _LANES))

    pltpu.async_copy(unif_hbm.at[pl.ds(pl.multiple_of(base, _LANES), _CHUNK)],
                     u_a, sem_ua)
    cp_deg.wait()

    def pick_half(u_ref, h):
        @plsc.parallel_loop(h * _NH, (h + 1) * _NH, unroll=7)
        def _pick(j):
            sl = pl.ds(j * _LANES, _LANES)
            cur = cur_v[sl]
            d0 = plsc.load_gather(deg_v, [cur])
            d = jnp.maximum(d0, 1)
            y = d.astype(jnp.float32) * u_ref[sl]
            i = y.astype(jnp.int32)
            idx = jnp.where(i.astype(jnp.float32) < y, i, i - 1)
            idx = jnp.maximum(jnp.minimum(idx, d - 1), 0)
            flat_v[sl] = cur * _MAX_DEG + idx
            d0_v[sl] = d0

    def sel_half(h):
        @plsc.parallel_loop(h * _NH, (h + 1) * _NH, unroll=7)
        def _sel(j):
            sl = pl.ds(j * _LANES, _LANES)
            g = gath_v[pl.ds(j * _LANES, _LANES)]
            cur_v[sl] = jnp.where(d0_v[sl] > 0, g, cur_v[sl])

    def one_step(t, u_ref, u_sem, first):
        pltpu.make_async_copy(
            unif_hbm.at[pl.ds(pl.multiple_of(base, _LANES), _CHUNK)],
            u_ref, u_sem).wait()
        pick_half(u_ref, 0)
        g0 = pltpu.async_copy(neigh_hbm.at[flat_v.at[pl.ds(0, _H)]],
                              gath_v.at[pl.ds(0, _H)], sem_g0)
        pick_half(u_ref, 1)
        g1 = pltpu.async_copy(neigh_hbm.at[flat_v.at[pl.ds(_H, _H)]],
                              gath_v.at[pl.ds(_H, _H)], sem_g1)
        g0.wait()
        if not first:
            pltpu.make_async_copy(
                cur_v,
                out_hbm.at[pl.ds(pl.multiple_of(base, _LANES), _CHUNK)],
                sem_out).wait()
        sel_half(0)
        g1.wait()
        sel_half(1)
        off = pl.multiple_of(t * _N + base, _LANES)
        pltpu.async_copy(cur_v, out_hbm.at[pl.ds(off, _CHUNK)], sem_out)

    def pair_body(k, carry):
        t0 = 2 * k
        t1 = 2 * k + 1
        off1 = pl.multiple_of(t1 * _N + base, _LANES)
        pltpu.async_copy(unif_hbm.at[pl.ds(off1, _CHUNK)], u_b, sem_ub)
        one_step(t0, u_a, sem_ua, False)

        @pl.when(k < _WALK_LEN // 2 - 1)
        def _():
            off2 = pl.multiple_of((t1 + 1) * _N + base, _LANES)
            pltpu.async_copy(unif_hbm.at[pl.ds(off2, _CHUNK)], u_a, sem_ua)

        one_step(t1, u_b, sem_ub, False)
        return carry

    off1 = pl.multiple_of(_N + base, _LANES)
    pltpu.async_copy(unif_hbm.at[pl.ds(off1, _CHUNK)], u_b, sem_ub)
    one_step(0, u_a, sem_ua, True)
    off2 = pl.multiple_of(2 * _N + base, _LANES)
    pltpu.async_copy(unif_hbm.at[pl.ds(off2, _CHUNK)], u_a, sem_ua)
    one_step(1, u_b, sem_ub, False)
    lax.fori_loop(1, _WALK_LEN // 2, pair_body, 0)

    pltpu.make_async_copy(
        cur_v, out_hbm.at[pl.ds(pl.multiple_of(base, _LANES), _CHUNK)],
        sem_out).wait()


@jax.jit
def kernel(neighbors, degrees, uniforms):
    mesh = plsc.VectorSubcoreMesh(core_axis_name="c", subcore_axis_name="s")
    walk = pl.kernel(
        _walk_body,
        out_type=jax.ShapeDtypeStruct((_WALK_LEN * _N,), jnp.int32),
        mesh=mesh,
        compiler_params=pltpu.CompilerParams(needs_layout_passes=False),
        scratch_types=[
            pltpu.VMEM((_N,), jnp.int32),
            pltpu.VMEM((_CHUNK,), jnp.int32),
            pltpu.VMEM((_CHUNK,), jnp.int32),
            pltpu.VMEM((_CHUNK,), jnp.int32),
            pltpu.VMEM((_CHUNK,), jnp.float32),
            pltpu.VMEM((_CHUNK,), jnp.float32),
            pltpu.VMEM((_CHUNK,), jnp.int32),
            pltpu.SemaphoreType.DMA,
            pltpu.SemaphoreType.DMA,
            pltpu.SemaphoreType.DMA,
            pltpu.SemaphoreType.DMA,
            pltpu.SemaphoreType.DMA,
            pltpu.SemaphoreType.DMA,
        ],
    )
    out = walk(neighbors.reshape(-1), degrees, uniforms.reshape(-1))
    return out.reshape(_WALK_LEN, _N)
